# Initial kernel scaffold; baseline (speedup 1.0000x reference)
import jax
import jax.numpy as jnp
from jax.experimental import pallas as pl


def kernel(xyz, new_xyz, features):
    raise NotImplementedError("write your pallas kernel here")


# speedup vs baseline: 2.7345x; 2.7345x over previous
import functools

import jax
import jax.numpy as jnp
from jax import lax
from jax.experimental import pallas as pl
from jax.experimental.pallas import tpu as pltpu
from jax.experimental.pallas import tpu_sc as plsc

KNN_NS = 32
VEC_RADIUS = 0.2
VEC_LENGTH = 0.1
VEC_NS = 32
P_TILE = 64
BIG_I = 2 ** 30


def _tc_body(xyzt_ref, new_ref, o_idx_ref, o_gx_ref,
             d_ref, ord_ref, gx_ref, gy_ref, gz_ref, idxs_ref):
    b = pl.program_id(0)
    n_pts = xyzt_ref.shape[2]
    pt = new_ref.shape[1]
    xyzt = xyzt_ref[0]
    new = new_ref[0]

    xr = xyzt[0:1, :]
    yr = xyzt[1:2, :]
    zr = xyzt[2:3, :]
    nx = new[:, 0:1]
    ny = new[:, 1:2]
    nz = new[:, 2:3]
    sum_x = (xr * xr + yr * yr) + zr * zr
    sum_q = (nx * nx + ny * ny) + nz * nz
    qx = lax.dot_general(new, xyzt, (((1,), (0,)), ((), ())))
    d_ref[...] = (sum_q + sum_x) - 2.0 * qx

    iota_n = lax.broadcasted_iota(jnp.int32, (pt, n_pts), 1)

    for s in range(KNN_NS):
        d = d_ref[...]
        m = jnp.min(d, axis=1, keepdims=True)
        cand = jnp.where(d == m, iota_n, BIG_I)
        i_s = jnp.min(cand, axis=1, keepdims=True)
        sel = iota_n == i_s
        oh = sel.astype(jnp.float32)
        p3 = lax.dot_general(oh, xyzt, (((1,), (1,)), ((), ())),
                             precision=lax.Precision.HIGHEST)
        gx_ref[:, s:s + 1] = p3[:, 0:1]
        gy_ref[:, s:s + 1] = p3[:, 1:2]
        gz_ref[:, s:s + 1] = p3[:, 2:3]
        d_ref[...] = jnp.where(sel, jnp.inf, d)

    newx = new[:, 0:1]
    newy = new[:, 1:2]
    newz = new[:, 2:3]

    cx = gx_ref[...] - newx
    cy = gy_ref[...] - newy
    cz = gz_ref[...] - newz
    mx = jnp.mean(cx, axis=1, keepdims=True)
    my = jnp.mean(cy, axis=1, keepdims=True)
    mz = jnp.mean(cz, axis=1, keepdims=True)
    tx = cx - mx
    ty = cy - my
    tz = cz - mz
    inv = 1.0 / (KNN_NS - 1)
    eye = (lax.broadcasted_iota(jnp.int32, (pt, pt), 0)
           == lax.broadcasted_iota(jnp.int32, (pt, pt), 1))

    def rowdot(a, b):
        m_full = lax.dot_general(a, b, (((1,), (1,)), ((), ())))
        return jnp.sum(jnp.where(eye, m_full, 0.0), axis=1, keepdims=True)

    axx = inv * rowdot(tx, tx)
    axy = inv * rowdot(tx, ty)
    axz = inv * rowdot(tx, tz)
    ayy = inv * rowdot(ty, ty)
    ayz = inv * rowdot(ty, tz)
    azz = inv * rowdot(tz, tz)

    def bf(x):
        return x.astype(jnp.bfloat16).astype(jnp.float32)

    vx = jnp.ones((pt, 1), jnp.float32)
    vy = jnp.ones((pt, 1), jnp.float32)
    vz = jnp.ones((pt, 1), jnp.float32)
    for it in range(20):
        if it == 0:
            cxx, cxy, cxz = bf(axx), bf(axy), bf(axz)
            cyy, cyz, czz = bf(ayy), bf(ayz), bf(azz)
        else:
            cxx, cxy, cxz = axx, axy, axz
            cyy, cyz, czz = ayy, ayz, azz
        mx_ = (cxx * vx + cxy * vy) + cxz * vz
        my_ = (cxy * vx + cyy * vy) + cyz * vz
        mz_ = (cxz * vx + cyz * vy) + czz * vz
        nrm = jnp.sqrt((mx_ * mx_ + my_ * my_) + mz_ * mz_)
        vx = mx_ / nrm
        vy = my_ / nrm
        vz = mz_ / nrm

    dvx = vx * VEC_LENGTH
    dvy = vy * VEC_LENGTH
    dvz = vz * VEC_LENGTH
    p1x = newx - dvx
    p1y = newy - dvy
    p1z = newz - dvz
    p2x = newx + dvx
    p2y = newy + dvy
    p2z = newz + dvz
    sx = p2x - p1x
    sy = p2y - p1y
    sz = p2z - p1z
    dd = (sx * sx + sy * sy) + sz * sz

    seg3 = jnp.concatenate([sx, sy, sz], axis=1)
    p13 = jnp.concatenate([p1x, p1y, p1z], axis=1)
    xs = lax.dot_general(seg3, xyzt, (((1,), (0,)), ((), ())))
    p1seg = (p1x * sx + p1y * sy) + p1z * sz
    proj = xs - p1seg
    tt = jnp.clip(proj / jnp.maximum(dd, 1e-12), 0.0, 1.0)
    xp1dot = lax.dot_general(p13, xyzt, (((1,), (0,)), ((), ())))
    pp1 = (p1x * p1x + p1y * p1y) + p1z * p1z
    xp1 = (sum_x - 2.0 * xp1dot) + pp1
    dist2 = (xp1 - 2.0 * tt * proj) + tt * tt * dd
    valid = dist2 <= VEC_RADIUS * VEC_RADIUS

    ord_ref[...] = jnp.where(valid, iota_n, BIG_I)
    for s in range(VEC_NS):
        o = ord_ref[...]
        i_s = jnp.min(o, axis=1, keepdims=True)
        idxs_ref[:, s:s + 1] = i_s
        ord_ref[...] = jnp.where(iota_n == i_s, BIG_I, o)

    raw = idxs_ref[...]
    first = jnp.where(raw[:, 0:1] < n_pts, raw[:, 0:1], 0)
    idx = jnp.where(raw < n_pts, raw, first)

    for s in range(VEC_NS):
        oh = (iota_n == idx[:, s:s + 1]).astype(jnp.float32)
        p3 = lax.dot_general(oh, xyzt, (((1,), (1,)), ((), ())),
                             precision=lax.Precision.HIGHEST)
        gx_ref[:, s:s + 1] = p3[:, 0:1]
        gy_ref[:, s:s + 1] = p3[:, 1:2]
        gz_ref[:, s:s + 1] = p3[:, 2:3]
    o_gx_ref[0, 0] = gx_ref[...] - newx
    o_gx_ref[0, 1] = gy_ref[...] - newy
    o_gx_ref[0, 2] = gz_ref[...] - newz
    o_idx_ref[0] = idx + b * n_pts


def _tc_pipeline(xyzt, new_xyz, interpret=False):
    b, _, n = xyzt.shape
    p = new_xyz.shape[1]
    grid = (b, p // P_TILE)
    return pl.pallas_call(
        _tc_body,
        grid=grid,
        in_specs=[
            pl.BlockSpec((1, 3, n), lambda i, j: (i, 0, 0)),
            pl.BlockSpec((1, P_TILE, 3), lambda i, j: (i, j, 0)),
        ],
        out_specs=[
            pl.BlockSpec((1, P_TILE, VEC_NS), lambda i, j: (i, j, 0)),
            pl.BlockSpec((1, 3, P_TILE, VEC_NS), lambda i, j: (i, 0, j, 0)),
        ],
        out_shape=[
            jax.ShapeDtypeStruct((b, p, VEC_NS), jnp.int32),
            jax.ShapeDtypeStruct((b, 3, p, VEC_NS), jnp.float32),
        ],
        compiler_params=pltpu.CompilerParams(
            vmem_limit_bytes=60 * 2 ** 20),
        scratch_shapes=[
            pltpu.VMEM((P_TILE, n), jnp.float32),
            pltpu.VMEM((P_TILE, n), jnp.int32),
            pltpu.VMEM((P_TILE, KNN_NS), jnp.float32),
            pltpu.VMEM((P_TILE, KNN_NS), jnp.float32),
            pltpu.VMEM((P_TILE, KNN_NS), jnp.float32),
            pltpu.VMEM((P_TILE, VEC_NS), jnp.int32),
        ],
        interpret=interpret,
    )(xyzt, new_xyz)


def _sc_gather(table, idx):
    m = idx.shape[0]
    d = table.shape[1]
    nw = 32
    ch = 512
    b_per_w = m // nw
    n_ch = b_per_w // ch
    mesh = plsc.VectorSubcoreMesh(core_axis_name="c", subcore_axis_name="s")

    @functools.partial(
        pl.kernel,
        mesh=mesh,
        out_type=jax.ShapeDtypeStruct((m, d), jnp.float32),
        compiler_params=pltpu.CompilerParams(use_tc_tiling_on_sc=False),
        scratch_types=[
            pltpu.VMEM((ch,), jnp.int32),
            pltpu.VMEM((ch, d), jnp.float32),
            pltpu.SemaphoreType.DMA,
        ],
    )
    def k(table_hbm, idx_hbm, out_hbm, idx_v, rows_v, sem):
        wid = lax.axis_index("s") * 2 + lax.axis_index("c")
        base = wid * b_per_w

        def body(c, carry):
            off = base + c * ch
            pltpu.sync_copy(idx_hbm.at[pl.ds(off, ch)], idx_v)
            pltpu.async_copy(table_hbm.at[idx_v], rows_v, sem).wait()
            pltpu.sync_copy(rows_v, out_hbm.at[pl.ds(off, ch)])
            return carry

        lax.fori_loop(0, n_ch, body, 0)

    return k(table, idx)


def kernel(xyz, new_xyz, features):
    b, n, _ = xyz.shape
    p = new_xyz.shape[1]
    c = features.shape[1]
    xyzt = jnp.swapaxes(xyz, 1, 2)
    idx_flat, gx = _tc_pipeline(xyzt, new_xyz)
    ft = jnp.swapaxes(features, 1, 2).reshape(b * n, c)
    rows = _sc_gather(ft, idx_flat.reshape(-1))
    gf = rows.reshape(b, p, VEC_NS, c).transpose(0, 3, 1, 2)
    return jnp.concatenate([gx, gf], axis=1)
